# baseline (speedup 1.0000x reference)
import functools

import jax
import jax.numpy as jnp
from jax import lax
from jax.experimental import pallas as pl
from jax.experimental.pallas import tpu as pltpu
from jax.experimental.pallas import tpu_sc as plsc

MARGIN = 0.5

SC_ROWS = 4096
NUM_WORKERS = 32
ROWS_PER_W = SC_ROWS // NUM_WORKERS
CH = 16
LANES = 16


def _tc_partials_kernel(x_ref, xr_ref, pc_ref, ps_ref, y_ref, out_ref, acc_ref,
                        *, grid_n, ce_steps, n_classes):
    i = pl.program_id(0)

    @pl.when(i == 0)
    def _init():
        acc_ref[0] = 0.0
        acc_ref[1] = 0.0
        acc_ref[2] = 0.0
        acc_ref[3] = 0.0

    x = x_ref[...]
    xr = xr_ref[...]
    rowsum = jnp.sum(x, axis=1)
    mask = rowsum != 0
    diff_rowsum = jnp.sum(jnp.abs(xr - x), axis=1)
    acc_ref[0] += jnp.sum(jnp.where(mask, diff_rowsum, 0.0))
    acc_ref[1] += jnp.sum(mask.astype(jnp.float32))

    @pl.when(i < ce_steps)
    def _ce():
        yv = y_ref[...]
        p_blk = yv.shape[0]
        lane = jax.lax.broadcasted_iota(jnp.int32, (p_blk, n_classes), 1)
        onehot = lane == yv
        for ref_, slot in ((pc_ref, 2), (ps_ref, 3)):
            logits = ref_[...]
            m = jnp.max(logits, axis=1, keepdims=True)
            lse = jnp.log(jnp.sum(jnp.exp(logits - m), axis=1)) + m[:, 0]
            picked = jnp.sum(jnp.where(onehot, logits, 0.0), axis=1)
            acc_ref[slot] += jnp.sum(lse - picked)

    @pl.when(i == grid_n - 1)
    def _fin():
        out_ref[0] = acc_ref[0]
        out_ref[1] = acc_ref[1]
        out_ref[2] = acc_ref[2]
        out_ref[3] = acc_ref[3]


def _sc_l1_body(x_hbm, xr_hbm, out_hbm, xb, xrb, ob, sem_x, sem_xr, *, d_model):
    c = lax.axis_index("c")
    s = lax.axis_index("s")
    wid = s * 2 + c
    base = wid * ROWS_PER_W
    n_chunks = ROWS_PER_W // CH
    zero = jnp.zeros((LANES,), jnp.float32)
    row_idx = lax.iota(jnp.int32, LANES)

    def chunk_loop(ci, carry):
        acc_abs, cnt = carry
        row0 = base + ci * CH
        cp_x = pltpu.async_copy(x_hbm.at[pl.ds(row0, CH)], xb, sem_x)
        cp_xr = pltpu.async_copy(xr_hbm.at[pl.ds(row0, CH)], xrb, sem_xr)
        cp_x.wait()
        cp_xr.wait()

        def col_loop(j, c2):
            rs, ad = c2
            col = jnp.full((LANES,), 0, jnp.int32) + j
            xv = plsc.load_gather(xb, [row_idx, col])
            xrv = plsc.load_gather(xrb, [row_idx, col])
            return rs + xv, ad + jnp.abs(xrv - xv)

        rs, ad = lax.fori_loop(0, d_model, col_loop, (zero, zero))
        mf = (rs != 0.0).astype(jnp.float32)
        return acc_abs + ad * mf, cnt + mf

    acc_abs, cnt = lax.fori_loop(0, n_chunks, chunk_loop, (zero, zero))
    ob[0, :] = acc_abs
    ob[1, :] = cnt
    pltpu.sync_copy(ob, out_hbm.at[wid])


def _sc_l1_partials(x2_sc, xr2_sc, d_model):
    mesh = plsc.VectorSubcoreMesh(core_axis_name="c", subcore_axis_name="s")
    run = functools.partial(
        pl.kernel,
        functools.partial(_sc_l1_body, d_model=d_model),
        out_type=jax.ShapeDtypeStruct((NUM_WORKERS, 2, LANES), jnp.float32),
        mesh=mesh,
        scratch_types=[
            pltpu.VMEM((CH, d_model), jnp.float32),
            pltpu.VMEM((CH, d_model), jnp.float32),
            pltpu.VMEM((2, LANES), jnp.float32),
            pltpu.SemaphoreType.DMA,
            pltpu.SemaphoreType.DMA,
        ],
        compiler_params=pltpu.CompilerParams(
            use_tc_tiling_on_sc=False, needs_layout_passes=False),
    )()
    return run(x2_sc, xr2_sc)


def kernel(stypred_cont, stypred_sty, pred_cont, pred_sty, x_rec, cont, sty, stylabels, y, x, D):
    B, S, Dm = x.shape
    N, C = pred_cont.shape
    R = B * S
    x2 = x.reshape(R, Dm)
    xr2 = x_rec.reshape(R, Dm)
    y2 = y.astype(jnp.int32).reshape(N, 1)

    tc_rows = R - SC_ROWS
    sc_part = _sc_l1_partials(x2[tc_rows:], xr2[tc_rows:], Dm)

    R_BLK = 1024
    grid_n = tc_rows // R_BLK
    ce_steps = 8
    P_BLK = N // ce_steps

    tc_part = pl.pallas_call(
        functools.partial(_tc_partials_kernel, grid_n=grid_n,
                          ce_steps=ce_steps, n_classes=C),
        grid=(grid_n,),
        in_specs=[
            pl.BlockSpec((R_BLK, Dm), lambda i: (i, 0)),
            pl.BlockSpec((R_BLK, Dm), lambda i: (i, 0)),
            pl.BlockSpec((P_BLK, C), lambda i: (i % 8, 0)),
            pl.BlockSpec((P_BLK, C), lambda i: (i % 8, 0)),
            pl.BlockSpec((P_BLK, 1), lambda i: (i % 8, 0)),
        ],
        out_specs=pl.BlockSpec(memory_space=pltpu.SMEM),
        out_shape=jax.ShapeDtypeStruct((4,), jnp.float32),
        scratch_shapes=[pltpu.SMEM((4,), jnp.float32)],
    )(x2[:tc_rows], xr2[:tc_rows], pred_cont, pred_sty, y2)

    sum_abs = tc_part[0] + jnp.sum(sc_part[:, 0, :])
    cnt = tc_part[1] + jnp.sum(sc_part[:, 1, :])
    inv_n = jnp.float32(1.0 / N)
    cls_cont = tc_part[2] * inv_n
    cls_sty = tc_part[3] * inv_n
    loss_rec = sum_abs / (cnt * jnp.float32(Dm)) + MARGIN
    loss = (cls_sty + cls_cont) * 0.5 + loss_rec
    zero = jnp.float32(0.0)
    return (loss, zero, zero, cls_cont, cls_sty, loss_rec)


# speedup vs baseline: 1.3092x; 1.3092x over previous
import functools

import jax
import jax.numpy as jnp
from jax import lax
from jax.experimental import pallas as pl
from jax.experimental.pallas import tpu as pltpu
from jax.experimental.pallas import tpu_sc as plsc

MARGIN = 0.5

SC_ROWS = 4096
NUM_WORKERS = 32
ROWS_PER_W = SC_ROWS // NUM_WORKERS
CH = 16
LANES = 16


def _tc_partials_kernel(x_ref, xr_ref, pc_ref, ps_ref, y_ref, out_ref, acc_ref,
                        *, grid_n, ce_steps, n_classes):
    i = pl.program_id(0)

    @pl.when(i == 0)
    def _init():
        acc_ref[0] = 0.0
        acc_ref[1] = 0.0
        acc_ref[2] = 0.0
        acc_ref[3] = 0.0

    x = x_ref[...]
    xr = xr_ref[...]
    rowsum = jnp.sum(x, axis=1)
    mask = rowsum != 0
    diff_rowsum = jnp.sum(jnp.abs(xr - x), axis=1)
    acc_ref[0] += jnp.sum(jnp.where(mask, diff_rowsum, 0.0))
    acc_ref[1] += jnp.sum(mask.astype(jnp.float32))

    @pl.when(i < ce_steps)
    def _ce():
        yv = y_ref[...]
        p_blk = yv.shape[0]
        lane = jax.lax.broadcasted_iota(jnp.int32, (p_blk, n_classes), 1)
        onehot = lane == yv
        for ref_, slot in ((pc_ref, 2), (ps_ref, 3)):
            logits = ref_[...]
            m = jnp.max(logits, axis=1, keepdims=True)
            lse = jnp.log(jnp.sum(jnp.exp(logits - m), axis=1)) + m[:, 0]
            picked = jnp.sum(jnp.where(onehot, logits, 0.0), axis=1)
            acc_ref[slot] += jnp.sum(lse - picked)

    @pl.when(i == grid_n - 1)
    def _fin():
        out_ref[0] = acc_ref[0]
        out_ref[1] = acc_ref[1]
        out_ref[2] = acc_ref[2]
        out_ref[3] = acc_ref[3]


def _sc_l1_body(x_hbm, xr_hbm, out_hbm,
                xb0, xb1, xrb0, xrb1, ob, sx0, sx1, sxr0, sxr1,
                *, d_model, base_row):
    c = lax.axis_index("c")
    s = lax.axis_index("s")
    wid = s * 2 + c
    base = base_row + wid * ROWS_PER_W
    n_chunks = ROWS_PER_W // CH
    zero = jnp.zeros((LANES,), jnp.float32)
    row_idx = lax.iota(jnp.int32, LANES)
    zero_i = jnp.zeros((LANES,), jnp.int32)

    xbs, xrbs = (xb0, xb1), (xrb0, xrb1)
    sxs, sxrs = (sx0, sx1), (sxr0, sxr1)

    def start(ci):
        b = ci % 2
        row0 = base + ci * CH
        return (pltpu.async_copy(x_hbm.at[pl.ds(row0, CH)], xbs[b], sxs[b]),
                pltpu.async_copy(xr_hbm.at[pl.ds(row0, CH)], xrbs[b], sxrs[b]))

    acc_abs = zero
    cnt = zero
    cps = start(0)
    for ci in range(n_chunks):
        nxt = start(ci + 1) if ci + 1 < n_chunks else None
        cps[0].wait()
        cps[1].wait()
        b = ci % 2
        xb, xrb = xbs[b], xrbs[b]

        def col_loop(j, c2, xb=xb, xrb=xrb):
            rs, ad = c2
            col = zero_i + j
            xv = plsc.load_gather(xb, [row_idx, col])
            xrv = plsc.load_gather(xrb, [row_idx, col])
            return rs + xv, ad + jnp.abs(xrv - xv)

        rs, ad = lax.fori_loop(0, d_model, col_loop, (zero, zero), unroll=8)
        mf = (rs != 0.0).astype(jnp.float32)
        acc_abs = acc_abs + ad * mf
        cnt = cnt + mf
        cps = nxt

    ob[0, :] = acc_abs
    ob[1, :] = cnt
    pltpu.sync_copy(ob, out_hbm.at[wid])


def _sc_l1_partials(x2, xr2, d_model, base_row):
    mesh = plsc.VectorSubcoreMesh(core_axis_name="c", subcore_axis_name="s")
    run = functools.partial(
        pl.kernel,
        functools.partial(_sc_l1_body, d_model=d_model, base_row=base_row),
        out_type=jax.ShapeDtypeStruct((NUM_WORKERS, 2, LANES), jnp.float32),
        mesh=mesh,
        scratch_types=[
            pltpu.VMEM((CH, d_model), jnp.float32),
            pltpu.VMEM((CH, d_model), jnp.float32),
            pltpu.VMEM((CH, d_model), jnp.float32),
            pltpu.VMEM((CH, d_model), jnp.float32),
            pltpu.VMEM((2, LANES), jnp.float32),
            pltpu.SemaphoreType.DMA,
            pltpu.SemaphoreType.DMA,
            pltpu.SemaphoreType.DMA,
            pltpu.SemaphoreType.DMA,
        ],
        compiler_params=pltpu.CompilerParams(
            use_tc_tiling_on_sc=True, needs_layout_passes=False),
    )()
    return run(x2, xr2)


def kernel(stypred_cont, stypred_sty, pred_cont, pred_sty, x_rec, cont, sty, stylabels, y, x, D):
    B, S, Dm = x.shape
    N, C = pred_cont.shape
    R = B * S
    x2 = x.reshape(R, Dm)
    xr2 = x_rec.reshape(R, Dm)
    y2 = y.astype(jnp.int32).reshape(N, 1)

    tc_rows = R - SC_ROWS
    sc_part = _sc_l1_partials(x2, xr2, Dm, tc_rows)

    R_BLK = 1024
    grid_n = tc_rows // R_BLK
    ce_steps = 8
    P_BLK = N // ce_steps

    tc_part = pl.pallas_call(
        functools.partial(_tc_partials_kernel, grid_n=grid_n,
                          ce_steps=ce_steps, n_classes=C),
        grid=(grid_n,),
        in_specs=[
            pl.BlockSpec((R_BLK, Dm), lambda i: (i, 0)),
            pl.BlockSpec((R_BLK, Dm), lambda i: (i, 0)),
            pl.BlockSpec((P_BLK, C), lambda i: (i % 8, 0)),
            pl.BlockSpec((P_BLK, C), lambda i: (i % 8, 0)),
            pl.BlockSpec((P_BLK, 1), lambda i: (i % 8, 0)),
        ],
        out_specs=pl.BlockSpec(memory_space=pltpu.SMEM),
        out_shape=jax.ShapeDtypeStruct((4,), jnp.float32),
        scratch_shapes=[pltpu.SMEM((4,), jnp.float32)],
    )(x2[:tc_rows], xr2[:tc_rows], pred_cont, pred_sty, y2)

    sum_abs = tc_part[0] + jnp.sum(sc_part[:, 0, :])
    cnt = tc_part[1] + jnp.sum(sc_part[:, 1, :])
    inv_n = jnp.float32(1.0 / N)
    cls_cont = tc_part[2] * inv_n
    cls_sty = tc_part[3] * inv_n
    loss_rec = sum_abs / (cnt * jnp.float32(Dm)) + MARGIN
    loss = (cls_sty + cls_cont) * 0.5 + loss_rec
    zero = jnp.float32(0.0)
    return (loss, zero, zero, cls_cont, cls_sty, loss_rec)


# speedup vs baseline: 1.7169x; 1.3114x over previous
import functools

import jax
import jax.numpy as jnp
from jax import lax
from jax.experimental import pallas as pl
from jax.experimental.pallas import tpu as pltpu
from jax.experimental.pallas import tpu_sc as plsc

MARGIN = 0.5

SC_ROWS = 4096
NUM_WORKERS = 32
ROWS_PER_W = SC_ROWS // NUM_WORKERS
CH = 16
LANES = 16


def _tc_partials_kernel(x_ref, xr_ref, pc_ref, ps_ref, y_ref, out_ref, acc_ref,
                        *, grid_n, ce_steps, n_classes):
    i = pl.program_id(0)

    @pl.when(i == 0)
    def _init():
        acc_ref[0] = 0.0
        acc_ref[1] = 0.0
        acc_ref[2] = 0.0
        acc_ref[3] = 0.0

    x = x_ref[...]
    xr = xr_ref[...]
    rowsum = jnp.sum(x, axis=1)
    mask = rowsum != 0
    diff_rowsum = jnp.sum(jnp.abs(xr - x), axis=1)
    acc_ref[0] += jnp.sum(jnp.where(mask, diff_rowsum, 0.0))
    acc_ref[1] += jnp.sum(mask.astype(jnp.float32))

    @pl.when(i < ce_steps)
    def _ce():
        yv = y_ref[...]
        p_blk = yv.shape[0]
        lane = jax.lax.broadcasted_iota(jnp.int32, (p_blk, n_classes), 1)
        onehot = lane == yv
        for ref_, slot in ((pc_ref, 2), (ps_ref, 3)):
            logits = ref_[...]
            m = jnp.max(logits, axis=1, keepdims=True)
            lse = jnp.log(jnp.sum(jnp.exp(logits - m), axis=1)) + m[:, 0]
            picked = jnp.sum(jnp.where(onehot, logits, 0.0), axis=1)
            acc_ref[slot] += jnp.sum(lse - picked)

    @pl.when(i == grid_n - 1)
    def _fin():
        out_ref[0] = acc_ref[0]
        out_ref[1] = acc_ref[1]
        out_ref[2] = acc_ref[2]
        out_ref[3] = acc_ref[3]


def _sc_l1_body(x_hbm, xr_hbm, out_hbm,
                xb0, xb1, xrb0, xrb1, ov, sx0, sx1, sxr0, sxr1,
                *, d_model, base_row):
    c = lax.axis_index("c")
    s = lax.axis_index("s")
    wid = s * 2 + c
    base = base_row + wid * ROWS_PER_W
    n_chunks = ROWS_PER_W // CH
    n_slices = d_model // LANES
    zero = jnp.zeros((LANES,), jnp.float32)

    xbs, xrbs = (xb0, xb1), (xrb0, xrb1)
    sxs, sxrs = (sx0, sx1), (sxr0, sxr1)

    def start(ci):
        b = ci % 2
        row0 = base + ci * CH
        return (pltpu.async_copy(x_hbm.at[pl.ds(row0, CH)], xbs[b], sxs[b]),
                pltpu.async_copy(xr_hbm.at[pl.ds(row0, CH)], xrbs[b], sxrs[b]))

    cps = start(0)
    for ci in range(n_chunks):
        nxt = start(ci + 1) if ci + 1 < n_chunks else None
        cps[0].wait()
        cps[1].wait()
        b = ci % 2
        xb, xrb = xbs[b], xrbs[b]

        def row_loop(r16, _, xb=xb, xrb=xrb, ci=ci):
            def col_loop(j, c2, xb=xb, xrb=xrb):
                rs, ad = c2
                xv = xb[r16, pl.ds(j * LANES, LANES)]
                xrv = xrb[r16, pl.ds(j * LANES, LANES)]
                return rs + xv, ad + jnp.abs(xrv - xv)

            rs, ad = lax.fori_loop(0, n_slices, col_loop, (zero, zero),
                                   unroll=8)
            r = ci * CH + r16
            orow = lax.shift_right_logical(r, 2)
            ocol = lax.shift_left(lax.bitwise_and(r, 3), 5)
            ov[orow, pl.ds(ocol, LANES)] = rs
            ov[orow, pl.ds(ocol + LANES, LANES)] = ad
            return 0

        lax.fori_loop(0, CH, row_loop, 0)
        cps = nxt

    pltpu.sync_copy(ov, out_hbm.at[pl.ds(wid * (ROWS_PER_W // 4),
                                         ROWS_PER_W // 4)])


def _sc_l1_partials(x2, xr2, d_model, base_row):
    mesh = plsc.VectorSubcoreMesh(core_axis_name="c", subcore_axis_name="s")
    run = functools.partial(
        pl.kernel,
        functools.partial(_sc_l1_body, d_model=d_model, base_row=base_row),
        out_type=jax.ShapeDtypeStruct((SC_ROWS // 4, 128), jnp.float32),
        mesh=mesh,
        scratch_types=[
            pltpu.VMEM((CH, d_model), jnp.float32),
            pltpu.VMEM((CH, d_model), jnp.float32),
            pltpu.VMEM((CH, d_model), jnp.float32),
            pltpu.VMEM((CH, d_model), jnp.float32),
            pltpu.VMEM((ROWS_PER_W // 4, 128), jnp.float32),
            pltpu.SemaphoreType.DMA,
            pltpu.SemaphoreType.DMA,
            pltpu.SemaphoreType.DMA,
            pltpu.SemaphoreType.DMA,
        ],
        compiler_params=pltpu.CompilerParams(
            use_tc_tiling_on_sc=True, needs_layout_passes=False),
    )()
    return run(x2, xr2)


def _tail_kernel(sc_ref, tcp_ref, out_ref, *, n_rows, d_model):
    b = sc_ref[...]
    sum_abs = tcp_ref[0]
    cnt = tcp_ref[1]
    for cpack in range(4):
        rs = jnp.sum(b[:, cpack * 32:cpack * 32 + 16], axis=1)
        ad = jnp.sum(b[:, cpack * 32 + 16:cpack * 32 + 32], axis=1)
        mask = rs != 0
        sum_abs += jnp.sum(jnp.where(mask, ad, 0.0))
        cnt += jnp.sum(mask.astype(jnp.float32))
    inv_n = 1.0 / jnp.float32(n_rows)
    cls_cont = tcp_ref[2] * inv_n
    cls_sty = tcp_ref[3] * inv_n
    loss_rec = sum_abs / (cnt * jnp.float32(d_model)) + MARGIN
    loss = (cls_sty + cls_cont) * 0.5 + loss_rec
    out_ref[0] = loss
    out_ref[1] = 0.0
    out_ref[2] = 0.0
    out_ref[3] = cls_cont
    out_ref[4] = cls_sty
    out_ref[5] = loss_rec


def kernel(stypred_cont, stypred_sty, pred_cont, pred_sty, x_rec, cont, sty, stylabels, y, x, D):
    B, S, Dm = x.shape
    N, C = pred_cont.shape
    R = B * S
    x2 = x.reshape(R, Dm)
    xr2 = x_rec.reshape(R, Dm)
    y2 = y.astype(jnp.int32).reshape(N, 1)

    tc_rows = R - SC_ROWS
    sc_part = _sc_l1_partials(x2, xr2, Dm, tc_rows)

    R_BLK = 1024
    grid_n = tc_rows // R_BLK
    ce_steps = 8
    P_BLK = N // ce_steps

    tc_part = pl.pallas_call(
        functools.partial(_tc_partials_kernel, grid_n=grid_n,
                          ce_steps=ce_steps, n_classes=C),
        grid=(grid_n,),
        in_specs=[
            pl.BlockSpec((R_BLK, Dm), lambda i: (i, 0)),
            pl.BlockSpec((R_BLK, Dm), lambda i: (i, 0)),
            pl.BlockSpec((P_BLK, C), lambda i: (i % 8, 0)),
            pl.BlockSpec((P_BLK, C), lambda i: (i % 8, 0)),
            pl.BlockSpec((P_BLK, 1), lambda i: (i % 8, 0)),
        ],
        out_specs=pl.BlockSpec(memory_space=pltpu.SMEM),
        out_shape=jax.ShapeDtypeStruct((4,), jnp.float32),
        scratch_shapes=[pltpu.SMEM((4,), jnp.float32)],
    )(x2[:tc_rows], xr2[:tc_rows], pred_cont, pred_sty, y2)

    out = pl.pallas_call(
        functools.partial(_tail_kernel, n_rows=N, d_model=Dm),
        in_specs=[
            pl.BlockSpec((SC_ROWS // 4, 128), lambda: (0, 0)),
            pl.BlockSpec(memory_space=pltpu.SMEM),
        ],
        out_specs=pl.BlockSpec(memory_space=pltpu.SMEM),
        out_shape=jax.ShapeDtypeStruct((6,), jnp.float32),
    )(sc_part, tc_part)

    return (out[0], out[1], out[2], out[3], out[4], out[5])


# speedup vs baseline: 3.2367x; 1.8852x over previous
import functools

import jax
import jax.numpy as jnp
from jax import lax
from jax.experimental import pallas as pl
from jax.experimental.pallas import tpu as pltpu
from jax.experimental.pallas import tpu_sc as plsc

MARGIN = 0.5

SC_ROWS = 4096
NUM_WORKERS = 32
ROWS_PER_W = SC_ROWS // NUM_WORKERS
CH = 16
LANES = 16


def _tc_partials_kernel(x_ref, xr_ref, pc_ref, ps_ref, y_ref, out_ref, acc_ref,
                        *, grid_n, ce_steps, n_classes):
    i = pl.program_id(0)

    @pl.when(i == 0)
    def _init():
        acc_ref[0] = 0.0
        acc_ref[1] = 0.0
        acc_ref[2] = 0.0
        acc_ref[3] = 0.0

    x = x_ref[...]
    xr = xr_ref[...]
    rowsum = jnp.sum(x, axis=1)
    mask = rowsum != 0
    diff_rowsum = jnp.sum(jnp.abs(xr - x), axis=1)
    acc_ref[0] += jnp.sum(jnp.where(mask, diff_rowsum, 0.0))
    acc_ref[1] += jnp.sum(mask.astype(jnp.float32))

    @pl.when(i < ce_steps)
    def _ce():
        yv = y_ref[...]
        p_blk = yv.shape[0]
        lane = jax.lax.broadcasted_iota(jnp.int32, (p_blk, n_classes), 1)
        onehot = lane == yv
        for ref_, slot in ((pc_ref, 2), (ps_ref, 3)):
            logits = ref_[...]
            m = jnp.max(logits, axis=1, keepdims=True)
            lse = jnp.log(jnp.sum(jnp.exp(logits - m), axis=1)) + m[:, 0]
            picked = jnp.sum(jnp.where(onehot, logits, 0.0), axis=1)
            acc_ref[slot] += jnp.sum(lse - picked)

    @pl.when(i == grid_n - 1)
    def _fin():
        out_ref[0] = acc_ref[0]
        out_ref[1] = acc_ref[1]
        out_ref[2] = acc_ref[2]
        out_ref[3] = acc_ref[3]


def _sc_l1_body(x_hbm, xr_hbm, out_hbm,
                xb0, xb1, xrb0, xrb1, ov, sx0, sx1, sxr0, sxr1,
                *, d_model, base_row):
    c = lax.axis_index("c")
    s = lax.axis_index("s")
    wid = s * 2 + c
    base = base_row + wid * ROWS_PER_W
    n_chunks = ROWS_PER_W // CH
    n_slices = d_model // LANES
    zero = jnp.zeros((LANES,), jnp.float32)

    xbs, xrbs = (xb0, xb1), (xrb0, xrb1)
    sxs, sxrs = (sx0, sx1), (sxr0, sxr1)

    def start(ci):
        b = ci % 2
        row0 = base + ci * CH
        return (pltpu.async_copy(x_hbm.at[pl.ds(row0, CH)], xbs[b], sxs[b]),
                pltpu.async_copy(xr_hbm.at[pl.ds(row0, CH)], xrbs[b], sxrs[b]))

    cps = start(0)
    for ci in range(n_chunks):
        nxt = start(ci + 1) if ci + 1 < n_chunks else None
        cps[0].wait()
        cps[1].wait()
        b = ci % 2
        xb, xrb = xbs[b], xrbs[b]

        def row_loop(r16, _, xb=xb, xrb=xrb, ci=ci):
            def col_loop(j, c2, xb=xb, xrb=xrb):
                rs, ad = c2
                xv = xb[r16, pl.ds(j * LANES, LANES)]
                xrv = xrb[r16, pl.ds(j * LANES, LANES)]
                return rs + xv, ad + jnp.abs(xrv - xv)

            rs, ad = lax.fori_loop(0, n_slices, col_loop, (zero, zero),
                                   unroll=8)
            r = ci * CH + r16
            orow = lax.shift_right_logical(r, 2)
            ocol = lax.shift_left(lax.bitwise_and(r, 3), 5)
            ov[orow, pl.ds(ocol, LANES)] = rs
            ov[orow, pl.ds(ocol + LANES, LANES)] = ad
            return 0

        lax.fori_loop(0, CH, row_loop, 0)
        cps = nxt

    pltpu.sync_copy(ov, out_hbm.at[pl.ds(wid * (ROWS_PER_W // 4),
                                         ROWS_PER_W // 4)])


def _sc_l1_partials(x2, xr2, d_model, base_row):
    mesh = plsc.VectorSubcoreMesh(core_axis_name="c", subcore_axis_name="s")
    run = functools.partial(
        pl.kernel,
        functools.partial(_sc_l1_body, d_model=d_model, base_row=base_row),
        out_type=jax.ShapeDtypeStruct((SC_ROWS // 4, 128), jnp.float32),
        mesh=mesh,
        scratch_types=[
            pltpu.VMEM((CH, d_model), jnp.float32),
            pltpu.VMEM((CH, d_model), jnp.float32),
            pltpu.VMEM((CH, d_model), jnp.float32),
            pltpu.VMEM((CH, d_model), jnp.float32),
            pltpu.VMEM((ROWS_PER_W // 4, 128), jnp.float32),
            pltpu.SemaphoreType.DMA,
            pltpu.SemaphoreType.DMA,
            pltpu.SemaphoreType.DMA,
            pltpu.SemaphoreType.DMA,
        ],
        compiler_params=pltpu.CompilerParams(
            use_tc_tiling_on_sc=True, needs_layout_passes=False),
    )()
    return run(x2, xr2)


def _tail_kernel(sc_ref, tcp_ref, out_ref, *, n_rows, d_model):
    b = sc_ref[...]
    sum_abs = tcp_ref[0]
    cnt = tcp_ref[1]
    for cpack in range(4):
        rs = jnp.sum(b[:, cpack * 32:cpack * 32 + 16], axis=1)
        ad = jnp.sum(b[:, cpack * 32 + 16:cpack * 32 + 32], axis=1)
        mask = rs != 0
        sum_abs += jnp.sum(jnp.where(mask, ad, 0.0))
        cnt += jnp.sum(mask.astype(jnp.float32))
    inv_n = 1.0 / jnp.float32(n_rows)
    cls_cont = tcp_ref[2] * inv_n
    cls_sty = tcp_ref[3] * inv_n
    loss_rec = sum_abs / (cnt * jnp.float32(d_model)) + MARGIN
    loss = (cls_sty + cls_cont) * 0.5 + loss_rec
    out_ref[0] = loss
    out_ref[1] = 0.0
    out_ref[2] = 0.0
    out_ref[3] = cls_cont
    out_ref[4] = cls_sty
    out_ref[5] = loss_rec


def kernel(stypred_cont, stypred_sty, pred_cont, pred_sty, x_rec, cont, sty, stylabels, y, x, D):
    B, S, Dm = x.shape
    N, C = pred_cont.shape
    R = B * S
    x2 = x.reshape(R, Dm)
    xr2 = x_rec.reshape(R, Dm)
    y2 = y.astype(jnp.int32).reshape(N, 1)

    tc_rows = R - SC_ROWS
    sc_part = _sc_l1_partials(x2, xr2, Dm, tc_rows)

    R_BLK = 1024
    grid_n = tc_rows // R_BLK
    ce_steps = 8
    P_BLK = N // ce_steps

    tc_part = pl.pallas_call(
        functools.partial(_tc_partials_kernel, grid_n=grid_n,
                          ce_steps=ce_steps, n_classes=C),
        grid=(grid_n,),
        in_specs=[
            pl.BlockSpec((R_BLK, Dm), lambda i: (i, 0)),
            pl.BlockSpec((R_BLK, Dm), lambda i: (i, 0)),
            pl.BlockSpec((P_BLK, C), lambda i: (i % 8, 0)),
            pl.BlockSpec((P_BLK, C), lambda i: (i % 8, 0)),
            pl.BlockSpec((P_BLK, 1), lambda i: (i % 8, 0)),
        ],
        out_specs=pl.BlockSpec(memory_space=pltpu.SMEM),
        out_shape=jax.ShapeDtypeStruct((4,), jnp.float32),
        scratch_shapes=[pltpu.SMEM((4,), jnp.float32)],
    )(x2, xr2, pred_cont, pred_sty, y2)

    out = pl.pallas_call(
        functools.partial(_tail_kernel, n_rows=N, d_model=Dm),
        in_specs=[
            pl.BlockSpec((SC_ROWS // 4, 128), lambda: (0, 0)),
            pl.BlockSpec(memory_space=pltpu.SMEM),
        ],
        out_specs=pl.BlockSpec(memory_space=pltpu.SMEM),
        out_shape=jax.ShapeDtypeStruct((6,), jnp.float32),
    )(sc_part, tc_part)

    return (out[0], out[1], out[2], out[3], out[4], out[5])


# speedup vs baseline: 5.0307x; 1.5542x over previous
import functools

import jax
import jax.numpy as jnp
from jax.experimental import pallas as pl
from jax.experimental.pallas import tpu as pltpu

MARGIN = 0.5


def _fused_loss_kernel(x_ref, xr_ref, pc_ref, ps_ref, y_ref, out_ref, acc_ref,
                       *, grid_n, ce_steps, n_rows, n_classes, d_model):
    i = pl.program_id(0)

    @pl.when(i == 0)
    def _init():
        acc_ref[0] = 0.0
        acc_ref[1] = 0.0
        acc_ref[2] = 0.0
        acc_ref[3] = 0.0

    x = x_ref[...]
    xr = xr_ref[...]
    rowsum = jnp.sum(x, axis=1)
    mask = rowsum != 0
    diff_rowsum = jnp.sum(jnp.abs(xr - x), axis=1)
    acc_ref[0] += jnp.sum(jnp.where(mask, diff_rowsum, 0.0))
    acc_ref[1] += jnp.sum(mask.astype(jnp.float32))

    @pl.when(i < ce_steps)
    def _ce():
        yv = y_ref[0, 0, :]
        p_blk = yv.shape[0]
        sub = jax.lax.broadcasted_iota(jnp.int32, (n_classes, p_blk), 0)
        onehot = sub == yv[None, :]
        for ref_, slot in ((pc_ref, 2), (ps_ref, 3)):
            logits = ref_[...]
            m = jnp.max(logits, axis=0, keepdims=True)
            lse = jnp.log(jnp.sum(jnp.exp(logits - m), axis=0)) + m[0]
            picked = jnp.sum(jnp.where(onehot, logits, 0.0), axis=0)
            acc_ref[slot] += jnp.sum(lse - picked)

    @pl.when(i == grid_n - 1)
    def _fin():
        inv_n = 1.0 / jnp.float32(n_rows)
        cls_cont = acc_ref[2] * inv_n
        cls_sty = acc_ref[3] * inv_n
        loss_rec = acc_ref[0] / (acc_ref[1] * jnp.float32(d_model)) + MARGIN
        loss = (cls_sty + cls_cont) * 0.5 + loss_rec
        out_ref[0] = loss
        out_ref[1] = 0.0
        out_ref[2] = 0.0
        out_ref[3] = cls_cont
        out_ref[4] = cls_sty
        out_ref[5] = loss_rec


def kernel(stypred_cont, stypred_sty, pred_cont, pred_sty, x_rec, cont, sty, stylabels, y, x, D):
    B, S, Dm = x.shape
    N, C = pred_cont.shape
    R = B * S
    x2 = x.reshape(R, Dm)
    xr2 = x_rec.reshape(R, Dm)

    R_BLK = 2048
    grid_n = R // R_BLK
    ce_steps = 8
    P_BLK = N // ce_steps
    pct = pred_cont.T
    pst = pred_sty.T
    y3 = y.astype(jnp.int32).reshape(ce_steps, 1, P_BLK)

    out = pl.pallas_call(
        functools.partial(_fused_loss_kernel, grid_n=grid_n, ce_steps=ce_steps,
                          n_rows=N, n_classes=C, d_model=Dm),
        grid=(grid_n,),
        in_specs=[
            pl.BlockSpec((R_BLK, Dm), lambda i: (i, 0)),
            pl.BlockSpec((R_BLK, Dm), lambda i: (i, 0)),
            pl.BlockSpec((C, P_BLK), lambda i: (0, i % 8)),
            pl.BlockSpec((C, P_BLK), lambda i: (0, i % 8)),
            pl.BlockSpec((1, 1, P_BLK), lambda i: (i % 8, 0, 0)),
        ],
        out_specs=pl.BlockSpec(memory_space=pltpu.SMEM),
        out_shape=jax.ShapeDtypeStruct((6,), jnp.float32),
        scratch_shapes=[pltpu.SMEM((4,), jnp.float32)],
    )(x2, xr2, pct, pst, y3)

    return (out[0], out[1], out[2], out[3], out[4], out[5])


# speedup vs baseline: 5.1951x; 1.0327x over previous
import functools

import jax
import jax.numpy as jnp
from jax.experimental import pallas as pl
from jax.experimental.pallas import tpu as pltpu

MARGIN = 0.5


def _fused_loss_kernel(x_ref, xr_ref, pc_ref, ps_ref, y_ref,
                       o_loss, o_zc, o_zs, o_cc, o_cs, o_rec, acc_ref,
                       *, grid_n, ce_steps, n_rows, n_classes, d_model):
    i = pl.program_id(0)

    @pl.when(i == 0)
    def _init():
        acc_ref[0] = 0.0
        acc_ref[1] = 0.0
        acc_ref[2] = 0.0
        acc_ref[3] = 0.0

    x = x_ref[...]
    xr = xr_ref[...]
    rowsum = jnp.sum(x, axis=1)
    mask = rowsum != 0
    diff_rowsum = jnp.sum(jnp.abs(xr - x), axis=1)
    acc_ref[0] += jnp.sum(jnp.where(mask, diff_rowsum, 0.0))
    acc_ref[1] += jnp.sum(mask.astype(jnp.float32))

    @pl.when(i < ce_steps)
    def _ce():
        yv = y_ref[0, 0, :]
        p_blk = yv.shape[0]
        sub = jax.lax.broadcasted_iota(jnp.int32, (n_classes, p_blk), 0)
        onehot = sub == yv[None, :]
        for ref_, slot in ((pc_ref, 2), (ps_ref, 3)):
            logits = ref_[...]
            m = jnp.max(logits, axis=0, keepdims=True)
            lse = jnp.log(jnp.sum(jnp.exp(logits - m), axis=0)) + m[0]
            picked = jnp.sum(jnp.where(onehot, logits, 0.0), axis=0)
            acc_ref[slot] += jnp.sum(lse - picked)

    @pl.when(i == grid_n - 1)
    def _fin():
        inv_n = 1.0 / jnp.float32(n_rows)
        cls_cont = acc_ref[2] * inv_n
        cls_sty = acc_ref[3] * inv_n
        loss_rec = acc_ref[0] / (acc_ref[1] * jnp.float32(d_model)) + MARGIN
        loss = (cls_sty + cls_cont) * 0.5 + loss_rec
        o_loss[0] = loss
        o_zc[0] = 0.0
        o_zs[0] = 0.0
        o_cc[0] = cls_cont
        o_cs[0] = cls_sty
        o_rec[0] = loss_rec


def kernel(stypred_cont, stypred_sty, pred_cont, pred_sty, x_rec, cont, sty, stylabels, y, x, D):
    B, S, Dm = x.shape
    N, C = pred_cont.shape
    R = B * S
    x2 = x.reshape(R, Dm)
    xr2 = x_rec.reshape(R, Dm)

    R_BLK = 2048
    grid_n = R // R_BLK
    ce_steps = 8
    P_BLK = N // ce_steps
    pct = pred_cont.T
    pst = pred_sty.T
    y3 = y.astype(jnp.int32).reshape(ce_steps, 1, P_BLK)

    out = pl.pallas_call(
        functools.partial(_fused_loss_kernel, grid_n=grid_n, ce_steps=ce_steps,
                          n_rows=N, n_classes=C, d_model=Dm),
        grid=(grid_n,),
        in_specs=[
            pl.BlockSpec((R_BLK, Dm), lambda i: (i, 0)),
            pl.BlockSpec((R_BLK, Dm), lambda i: (i, 0)),
            pl.BlockSpec((C, P_BLK), lambda i: (0, i % 8)),
            pl.BlockSpec((C, P_BLK), lambda i: (0, i % 8)),
            pl.BlockSpec((1, 1, P_BLK), lambda i: (i % 8, 0, 0)),
        ],
        out_specs=[pl.BlockSpec(memory_space=pltpu.SMEM)] * 6,
        out_shape=[jax.ShapeDtypeStruct((1,), jnp.float32)] * 6,
        scratch_shapes=[pltpu.SMEM((4,), jnp.float32)],
    )(x2, xr2, pct, pst, y3)

    return tuple(o.reshape(()) for o in out)
